# Initial kernel scaffold; baseline (speedup 1.0000x reference)
import jax
import jax.numpy as jnp
from jax.experimental import pallas as pl


def kernel(z, y, motif_vector):
    raise NotImplementedError("write your pallas kernel here")


# speedup vs baseline: 2.8032x; 2.8032x over previous
import functools

import jax
import jax.numpy as jnp
from jax.experimental import pallas as pl
from jax.experimental.pallas import tpu as pltpu

_B = 16384
_NH = 256
_NM = 1024
_NMPC = 8
_TEMP = 0.2
_EPS = 1e-4

_BLK = 256
_NBLK = _B // _BLK


def _loss_kernel(z_ref, y_ref, m_ref, acc_ref, msq_ref):
    i = pl.program_id(0)

    @pl.when(i == 0)
    def _():
        m = m_ref[...]
        msq_ref[...] = jnp.sum(m * m, axis=1, keepdims=True).T

    zb = z_ref[...]
    zsq = jnp.sum(zb * zb, axis=1, keepdims=True)
    xp = jax.lax.dot_general(
        zb, m_ref[...], (((1,), (1,)), ((), ())),
        preferred_element_type=jnp.float32,
    )
    d = zsq + msq_ref[...] - 2.0 * xp
    r = (d + 1.0) / (d + _EPS)
    r2 = r * r
    sim = r2 * r2 * r
    tot = jnp.sum(sim, axis=1)
    cls = jax.lax.broadcasted_iota(jnp.int32, (_BLK, _NM), 1) >> 3
    yb = y_ref[0, 0, :]
    pos = jnp.sum(jnp.where(cls == yb[:, None], sim, 0.0), axis=1)
    part = jnp.sum(jnp.log(tot) - jnp.log(pos)).reshape(1, 1)

    @pl.when(i == 0)
    def _():
        acc_ref[...] = part

    @pl.when(i != 0)
    def _():
        acc_ref[...] += part


def kernel(z, y, motif_vector):
    y3 = y.reshape(_NBLK, 1, _BLK)
    acc = pl.pallas_call(
        _loss_kernel,
        grid=(_NBLK,),
        in_specs=[
            pl.BlockSpec((_BLK, _NH), lambda i: (i, 0)),
            pl.BlockSpec((1, 1, _BLK), lambda i: (i, 0, 0)),
            pl.BlockSpec((_NM, _NH), lambda i: (0, 0)),
        ],
        out_specs=pl.BlockSpec((1, 1), lambda i: (0, 0)),
        out_shape=jax.ShapeDtypeStruct((1, 1), jnp.float32),
        scratch_shapes=[pltpu.VMEM((1, _NM), jnp.float32)],
    )(z, y3, motif_vector)
    return acc[0, 0] / _B
